# baseline (speedup 1.0000x reference)
import functools

import jax
import jax.numpy as jnp
from jax import lax
from jax.experimental import pallas as pl
from jax.experimental.pallas import tpu as pltpu
from jax.experimental.pallas import tpu_sc as plsc

N_NODES = 10000
NP = 10240
D = 128
DH = D // 2
NSUB = 16
G = 256


_MESH = plsc.VectorSubcoreMesh(core_axis_name="c", subcore_axis_name="s")
_SC_PARAMS = pltpu.CompilerParams(needs_layout_passes=False,
                                  use_tc_tiling_on_sc=False)


def _deg_body(dst_hbm, out_hbm, dstS, hist, stag, res, acc):
    c = lax.axis_index("c")
    s = lax.axis_index("s")
    wid = c * NSUB + s
    ept = dstS.shape[0]

    pltpu.sync_copy(dst_hbm.at[pl.ds(wid * ept, ept)], dstS)

    zero16 = jnp.zeros((16,), jnp.float32)

    def _zhist(i, _):
        hist[pl.ds(i * 16, 16)] = zero16
        return 0

    lax.fori_loop(0, NP // 16, _zhist, 0)

    ones16 = jnp.ones((16,), jnp.float32)

    def _histb(i, _):
        dv = dstS[pl.ds(i * 16, 16)]
        plsc.addupdate_scatter(hist, [dv], ones16)
        return 0

    lax.fori_loop(0, ept // 16, _histb, 0)

    pltpu.sync_copy(hist, acc.at[s])
    plsc.subcore_barrier()

    npt = NP // NSUB
    base = s * npt

    def _zres(i, _):
        res[pl.ds(i * 16, 16)] = zero16
        return 0

    lax.fori_loop(0, npt // 16, _zres, 0)
    for t in range(NSUB):
        pltpu.sync_copy(acc.at[t, pl.ds(base, npt)], stag)

        def _acc(i, _):
            res[pl.ds(i * 16, 16)] += stag[pl.ds(i * 16, 16)]
            return 0

        lax.fori_loop(0, npt // 16, _acc, 0)

    pltpu.sync_copy(res, out_hbm.at[c, pl.ds(base, npt)])


def _deg_call(dst_flat, ept):
    kfn = functools.partial(
        pl.kernel,
        mesh=_MESH,
        out_type=jax.ShapeDtypeStruct((2, NP), jnp.float32),
        scratch_types=[
            pltpu.VMEM((ept,), jnp.int32),
            pltpu.VMEM((NP,), jnp.float32),
            pltpu.VMEM((NP // NSUB,), jnp.float32),
            pltpu.VMEM((NP // NSUB,), jnp.float32),
            pltpu.VMEM_SHARED((NSUB, NP), jnp.float32),
        ],
        compiler_params=_SC_PARAMS,
    )(_deg_body)
    return kfn(dst_flat)


def _agg_body(y_hbm, src_hbm, dst_hbm, out_hbm, srcS, dstS, buf0, buf1, sem0, sem1, acc):
    c = lax.axis_index("c")
    s = lax.axis_index("s")
    ng = srcS.shape[0]
    rows_per_tile = NP // NSUB
    wb = 128
    nchunk = rows_per_tile // wb
    bufs = (buf0, buf1)
    sems = (sem0, sem1)

    zero16 = jnp.zeros((16,), jnp.float32)

    def _zb(i, _):
        buf0[i, pl.ds(0, 16)] = zero16
        buf0[i, pl.ds(16, 16)] = zero16
        buf0[i, pl.ds(32, 16)] = zero16
        buf0[i, pl.ds(48, 16)] = zero16
        return 0

    lax.fori_loop(0, wb, _zb, 0)
    for k in range(nchunk):
        pltpu.sync_copy(buf0.at[pl.ds(0, wb)],
                        acc.at[pl.ds(s * rows_per_tile + k * wb, wb)])

    pltpu.sync_copy(src_hbm.at[s], srcS)
    pltpu.sync_copy(dst_hbm.at[s], dstS)
    plsc.subcore_barrier()

    def _body(t, _):
        j0 = 2 * t
        j1 = 2 * t + 1
        cp0 = pltpu.async_copy(y_hbm.at[c].at[srcS.at[j0]], buf0, sem0)
        cp1 = pltpu.async_copy(y_hbm.at[c].at[srcS.at[j1]], buf1, sem1)
        cp0.wait()
        pltpu.sync_copy(buf0, acc.at[dstS.at[j0]], add=True)
        cp1.wait()
        pltpu.sync_copy(buf1, acc.at[dstS.at[j1]], add=True)
        return 0

    lax.fori_loop(0, ng // 2, _body, 0)

    plsc.subcore_barrier()
    for k in range(nchunk):
        pltpu.sync_copy(acc.at[pl.ds(s * rows_per_tile + k * wb, wb)],
                        buf0.at[pl.ds(0, wb)])
        pltpu.sync_copy(buf0.at[pl.ds(0, wb)],
                        out_hbm.at[c, pl.ds(s * rows_per_tile + k * wb, wb)])


def _agg_call(y, src3, dst3, ng):
    kfn = functools.partial(
        pl.kernel,
        mesh=_MESH,
        out_type=jax.ShapeDtypeStruct((2, NP, DH), jnp.float32),
        scratch_types=[
            pltpu.VMEM((ng, G), jnp.int32),
            pltpu.VMEM((ng, G), jnp.int32),
            pltpu.VMEM((G, DH), jnp.float32),
            pltpu.VMEM((G, DH), jnp.float32),
            pltpu.SemaphoreType.DMA,
            pltpu.SemaphoreType.DMA,
            pltpu.VMEM_SHARED((NP, DH), jnp.float32),
        ],
        compiler_params=_SC_PARAMS,
    )(_agg_body)
    return kfn(y, src3, dst3)


_RB = 1280


def _dinv_block(degp):
    deg = degp[0] + degp[1]
    return lax.rsqrt(jnp.maximum(deg, 1.0))


def _split_store(o_ref, y):
    o_ref[0] = y[:, :DH]
    o_ref[1] = y[:, DH:]


def _k1_body(x_ref, w_ref, degp_ref, o_ref):
    dinv = _dinv_block(degp_ref[...])
    y = jnp.dot(x_ref[...], w_ref[...], preferred_element_type=jnp.float32)
    _split_store(o_ref, y * dinv)


def _k1_call(x, w1, degp):
    grid = NP // _RB
    return pl.pallas_call(
        _k1_body,
        grid=(grid,),
        in_specs=[
            pl.BlockSpec((_RB, D), lambda i: (i, 0)),
            pl.BlockSpec((D, D), lambda i: (0, 0)),
            pl.BlockSpec((2, _RB, 1), lambda i: (0, i, 0)),
        ],
        out_specs=pl.BlockSpec((2, _RB, DH), lambda i: (0, i, 0)),
        out_shape=jax.ShapeDtypeStruct((2, NP, DH), jnp.float32),
    )(x, w1, degp)


def _k2_body(p_ref, degp_ref, b_ref, w_ref, o_ref):
    i = pl.program_id(0)
    dinv = _dinv_block(degp_ref[...])
    sfull = jnp.concatenate([p_ref[0], p_ref[1]], axis=1)
    h = jax.nn.relu(sfull * dinv + b_ref[...])
    y = jnp.dot(h, w_ref[...], preferred_element_type=jnp.float32) * dinv
    rows = i * _RB + lax.broadcasted_iota(jnp.int32, (_RB, 1), 0)
    _split_store(o_ref, jnp.where(rows < N_NODES, y, 0.0))


def _k2_call(p, degp, b1, w2):
    grid = NP // _RB
    return pl.pallas_call(
        _k2_body,
        grid=(grid,),
        in_specs=[
            pl.BlockSpec((2, _RB, DH), lambda i: (0, i, 0)),
            pl.BlockSpec((2, _RB, 1), lambda i: (0, i, 0)),
            pl.BlockSpec((1, D), lambda i: (0, 0)),
            pl.BlockSpec((D, D), lambda i: (0, 0)),
        ],
        out_specs=pl.BlockSpec((2, _RB, DH), lambda i: (0, i, 0)),
        out_shape=jax.ShapeDtypeStruct((2, NP, DH), jnp.float32),
    )(p, degp, b1, w2)


def _k3_body(p_ref, degp_ref, b2_ref, wcat_ref, bcat_ref, wsm_ref,
             bsm_ref, big_ref, small_ref, gacc_ref):
    i = pl.program_id(0)
    nsteps = pl.num_programs(0)
    dinv = _dinv_block(degp_ref[...])
    sfull = jnp.concatenate([p_ref[0], p_ref[1]], axis=1)
    h = jax.nn.relu(sfull * dinv + b2_ref[...])
    rows = i * _RB + lax.broadcasted_iota(jnp.int32, (_RB, 1), 0)
    hm = jnp.where(rows < N_NODES, h, 0.0)

    @pl.when(i == 0)
    def _():
        gacc_ref[...] = jnp.zeros_like(gacc_ref)

    gacc_ref[...] += jnp.sum(hm, axis=0, keepdims=True)

    big_ref[...] = (
        jnp.dot(h, wcat_ref[...], preferred_element_type=jnp.float32)
        + bcat_ref[...]
    )

    @pl.when(i == nsteps - 1)
    def _():
        g = gacc_ref[...] * (1.0 / N_NODES)
        small_ref[...] = (
            jnp.dot(g, wsm_ref[...], preferred_element_type=jnp.float32)
            + bsm_ref[...]
        )


def _k3_call(p, degp, b2, wcat, bcat, wsm, bsm):
    grid = NP // _RB
    so = wcat.shape[1]
    sm = wsm.shape[1]
    return pl.pallas_call(
        _k3_body,
        grid=(grid,),
        in_specs=[
            pl.BlockSpec((2, _RB, DH), lambda i: (0, i, 0)),
            pl.BlockSpec((2, _RB, 1), lambda i: (0, i, 0)),
            pl.BlockSpec((1, D), lambda i: (0, 0)),
            pl.BlockSpec((D, so), lambda i: (0, 0)),
            pl.BlockSpec((1, so), lambda i: (0, 0)),
            pl.BlockSpec((D, sm), lambda i: (0, 0)),
            pl.BlockSpec((1, sm), lambda i: (0, 0)),
        ],
        out_specs=[
            pl.BlockSpec((_RB, so), lambda i: (i, 0)),
            pl.BlockSpec((1, sm), lambda i: (0, 0)),
        ],
        out_shape=[
            jax.ShapeDtypeStruct((NP, so), jnp.float32),
            jax.ShapeDtypeStruct((1, sm), jnp.float32),
        ],
        scratch_shapes=[pltpu.VMEM((1, D), jnp.float32)],
    )(p, degp, b2, wcat, bcat, wsm, bsm)


def kernel(node_features, edge_index, W1, b1, W2, b2, Wc, bc, Wh, bh, Wt, bt,
           Wp1, bp1, Wp2, bp2, Wd, bd, Ws, bs):
    n, d = node_features.shape
    e = edge_index.shape[1]
    per = NSUB * 2 * G
    etot = ((e + n + per - 1) // per) * per
    ng = etot // (NSUB * G)

    ei = edge_index.astype(jnp.int32)
    loop = jnp.arange(n, dtype=jnp.int32)
    padv = jnp.full((etot - e - n,), NP - 1, jnp.int32)
    src = jnp.concatenate([ei[0], loop, padv])
    dst = jnp.concatenate([ei[1], loop, padv])
    src3 = src.reshape(NSUB, ng, G)
    dst3 = dst.reshape(NSUB, ng, G)

    x = jnp.pad(node_features, ((0, NP - n), (0, 0)))

    ept_deg = etot // 32

    degp = _deg_call(dst, ept_deg)
    degp = degp.reshape(2, NP, 1)

    y1 = _k1_call(x, W1, degp)
    s1 = _agg_call(y1, src3, dst3, ng)
    y2 = _k2_call(s1, degp, b1.reshape(1, -1), W2)
    s2 = _agg_call(y2, src3, dst3, ng)

    wcat = jnp.concatenate([Wp1, Wp2, Wd], axis=1)
    bcat = jnp.concatenate([bp1, bp2, bd]).reshape(1, -1)
    wsm = jnp.concatenate([Wc, Wh, Wt, Ws], axis=1)
    bsm = jnp.concatenate([bc, bh, bt, bs]).reshape(1, -1)

    big, small = _k3_call(s2, degp, b2.reshape(1, -1), wcat, bcat, wsm, bsm)

    s = Wp1.shape[1]
    p1 = big[:n, :s]
    p2 = big[:n, s:2 * s]
    dep = big[:n, 2 * s:3 * s]
    value = small[:, :1]
    high = small[:, 1:5]
    mtype = small[:, 5:8]
    sel = small[:, 8:]
    return (value, high, mtype, p1, p2, dep, sel)


# speedup vs baseline: 1.0065x; 1.0065x over previous
import functools

import jax
import jax.numpy as jnp
from jax import lax
from jax.experimental import pallas as pl
from jax.experimental.pallas import tpu as pltpu
from jax.experimental.pallas import tpu_sc as plsc

N_NODES = 10000
NP = 10240
D = 128
DH = D // 2
NSUB = 16
G = 128


_MESH = plsc.VectorSubcoreMesh(core_axis_name="c", subcore_axis_name="s")
_SC_PARAMS = pltpu.CompilerParams(needs_layout_passes=False,
                                  use_tc_tiling_on_sc=False)


def _deg_body(dst_hbm, out_hbm, dstS, hist, stag, res, acc):
    c = lax.axis_index("c")
    s = lax.axis_index("s")
    wid = c * NSUB + s
    ept = dstS.shape[0]

    pltpu.sync_copy(dst_hbm.at[pl.ds(wid * ept, ept)], dstS)

    zero16 = jnp.zeros((16,), jnp.float32)

    def _zhist(i, _):
        hist[pl.ds(i * 16, 16)] = zero16
        return 0

    lax.fori_loop(0, NP // 16, _zhist, 0)

    ones16 = jnp.ones((16,), jnp.float32)

    def _histb(i, _):
        dv = dstS[pl.ds(i * 16, 16)]
        plsc.addupdate_scatter(hist, [dv], ones16)
        return 0

    lax.fori_loop(0, ept // 16, _histb, 0)

    pltpu.sync_copy(hist, acc.at[s])
    plsc.subcore_barrier()

    npt = NP // NSUB
    base = s * npt

    def _zres(i, _):
        res[pl.ds(i * 16, 16)] = zero16
        return 0

    lax.fori_loop(0, npt // 16, _zres, 0)
    for t in range(NSUB):
        pltpu.sync_copy(acc.at[t, pl.ds(base, npt)], stag)

        def _acc(i, _):
            res[pl.ds(i * 16, 16)] += stag[pl.ds(i * 16, 16)]
            return 0

        lax.fori_loop(0, npt // 16, _acc, 0)

    pltpu.sync_copy(res, out_hbm.at[c, pl.ds(base, npt)])


def _deg_call(dst_flat, ept):
    kfn = functools.partial(
        pl.kernel,
        mesh=_MESH,
        out_type=jax.ShapeDtypeStruct((2, NP), jnp.float32),
        scratch_types=[
            pltpu.VMEM((ept,), jnp.int32),
            pltpu.VMEM((NP,), jnp.float32),
            pltpu.VMEM((NP // NSUB,), jnp.float32),
            pltpu.VMEM((NP // NSUB,), jnp.float32),
            pltpu.VMEM_SHARED((NSUB, NP), jnp.float32),
        ],
        compiler_params=_SC_PARAMS,
    )(_deg_body)
    return kfn(dst_flat)


def _agg_body(y_hbm, src_hbm, dst_hbm, out_hbm, srcS, dstS,
              buf0, buf1, buf2, buf3, sem0, sem1, sem2, sem3, acc):
    c = lax.axis_index("c")
    s = lax.axis_index("s")
    ng = srcS.shape[0]
    rows_per_tile = NP // NSUB
    wb = 128
    nchunk = rows_per_tile // wb
    bufs = (buf0, buf1, buf2, buf3)
    sems = (sem0, sem1, sem2, sem3)

    zero16 = jnp.zeros((16,), jnp.float32)

    def _zb(i, _):
        buf0[i, pl.ds(0, 16)] = zero16
        buf0[i, pl.ds(16, 16)] = zero16
        buf0[i, pl.ds(32, 16)] = zero16
        buf0[i, pl.ds(48, 16)] = zero16
        return 0

    lax.fori_loop(0, wb, _zb, 0)
    for k in range(nchunk):
        pltpu.sync_copy(buf0.at[pl.ds(0, wb)],
                        acc.at[pl.ds(s * rows_per_tile + k * wb, wb)])

    pltpu.sync_copy(src_hbm.at[s], srcS)
    pltpu.sync_copy(dst_hbm.at[s], dstS)
    plsc.subcore_barrier()

    def _body(t, _):
        cps = [pltpu.async_copy(y_hbm.at[c].at[srcS.at[4 * t + b]],
                                bufs[b], sems[b])
               for b in range(4)]
        for b in range(4):
            cps[b].wait()
            pltpu.sync_copy(bufs[b], acc.at[dstS.at[4 * t + b]], add=True)
        return 0

    lax.fori_loop(0, ng // 4, _body, 0)

    plsc.subcore_barrier()
    for k in range(nchunk):
        pltpu.sync_copy(acc.at[pl.ds(s * rows_per_tile + k * wb, wb)],
                        buf0.at[pl.ds(0, wb)])
        pltpu.sync_copy(buf0.at[pl.ds(0, wb)],
                        out_hbm.at[c, pl.ds(s * rows_per_tile + k * wb, wb)])


def _agg_call(y, src3, dst3, ng):
    kfn = functools.partial(
        pl.kernel,
        mesh=_MESH,
        out_type=jax.ShapeDtypeStruct((2, NP, DH), jnp.float32),
        scratch_types=[
            pltpu.VMEM((ng, G), jnp.int32),
            pltpu.VMEM((ng, G), jnp.int32),
            pltpu.VMEM((G, DH), jnp.float32),
            pltpu.VMEM((G, DH), jnp.float32),
            pltpu.VMEM((G, DH), jnp.float32),
            pltpu.VMEM((G, DH), jnp.float32),
            pltpu.SemaphoreType.DMA,
            pltpu.SemaphoreType.DMA,
            pltpu.SemaphoreType.DMA,
            pltpu.SemaphoreType.DMA,
            pltpu.VMEM_SHARED((NP, DH), jnp.float32),
        ],
        compiler_params=_SC_PARAMS,
    )(_agg_body)
    return kfn(y, src3, dst3)


_RB = 1280


def _dinv_block(degp):
    deg = degp[0] + degp[1]
    return lax.rsqrt(jnp.maximum(deg, 1.0))


def _split_store(o_ref, y):
    o_ref[0] = y[:, :DH]
    o_ref[1] = y[:, DH:]


def _k1_body(x_ref, w_ref, degp_ref, o_ref):
    dinv = _dinv_block(degp_ref[...])
    y = jnp.dot(x_ref[...], w_ref[...], preferred_element_type=jnp.float32)
    _split_store(o_ref, y * dinv)


def _k1_call(x, w1, degp):
    grid = NP // _RB
    return pl.pallas_call(
        _k1_body,
        grid=(grid,),
        in_specs=[
            pl.BlockSpec((_RB, D), lambda i: (i, 0)),
            pl.BlockSpec((D, D), lambda i: (0, 0)),
            pl.BlockSpec((2, _RB, 1), lambda i: (0, i, 0)),
        ],
        out_specs=pl.BlockSpec((2, _RB, DH), lambda i: (0, i, 0)),
        out_shape=jax.ShapeDtypeStruct((2, NP, DH), jnp.float32),
    )(x, w1, degp)


def _k2_body(p_ref, degp_ref, b_ref, w_ref, o_ref):
    i = pl.program_id(0)
    dinv = _dinv_block(degp_ref[...])
    sfull = jnp.concatenate([p_ref[0], p_ref[1]], axis=1)
    h = jax.nn.relu(sfull * dinv + b_ref[...])
    y = jnp.dot(h, w_ref[...], preferred_element_type=jnp.float32) * dinv
    rows = i * _RB + lax.broadcasted_iota(jnp.int32, (_RB, 1), 0)
    _split_store(o_ref, jnp.where(rows < N_NODES, y, 0.0))


def _k2_call(p, degp, b1, w2):
    grid = NP // _RB
    return pl.pallas_call(
        _k2_body,
        grid=(grid,),
        in_specs=[
            pl.BlockSpec((2, _RB, DH), lambda i: (0, i, 0)),
            pl.BlockSpec((2, _RB, 1), lambda i: (0, i, 0)),
            pl.BlockSpec((1, D), lambda i: (0, 0)),
            pl.BlockSpec((D, D), lambda i: (0, 0)),
        ],
        out_specs=pl.BlockSpec((2, _RB, DH), lambda i: (0, i, 0)),
        out_shape=jax.ShapeDtypeStruct((2, NP, DH), jnp.float32),
    )(p, degp, b1, w2)


def _k3_body(p_ref, degp_ref, b2_ref, wcat_ref, bcat_ref, wsm_ref,
             bsm_ref, big_ref, small_ref, gacc_ref):
    i = pl.program_id(0)
    nsteps = pl.num_programs(0)
    dinv = _dinv_block(degp_ref[...])
    sfull = jnp.concatenate([p_ref[0], p_ref[1]], axis=1)
    h = jax.nn.relu(sfull * dinv + b2_ref[...])
    rows = i * _RB + lax.broadcasted_iota(jnp.int32, (_RB, 1), 0)
    hm = jnp.where(rows < N_NODES, h, 0.0)

    @pl.when(i == 0)
    def _():
        gacc_ref[...] = jnp.zeros_like(gacc_ref)

    gacc_ref[...] += jnp.sum(hm, axis=0, keepdims=True)

    big_ref[...] = (
        jnp.dot(h, wcat_ref[...], preferred_element_type=jnp.float32)
        + bcat_ref[...]
    )

    @pl.when(i == nsteps - 1)
    def _():
        g = gacc_ref[...] * (1.0 / N_NODES)
        small_ref[...] = (
            jnp.dot(g, wsm_ref[...], preferred_element_type=jnp.float32)
            + bsm_ref[...]
        )


def _k3_call(p, degp, b2, wcat, bcat, wsm, bsm):
    grid = NP // _RB
    so = wcat.shape[1]
    sm = wsm.shape[1]
    return pl.pallas_call(
        _k3_body,
        grid=(grid,),
        in_specs=[
            pl.BlockSpec((2, _RB, DH), lambda i: (0, i, 0)),
            pl.BlockSpec((2, _RB, 1), lambda i: (0, i, 0)),
            pl.BlockSpec((1, D), lambda i: (0, 0)),
            pl.BlockSpec((D, so), lambda i: (0, 0)),
            pl.BlockSpec((1, so), lambda i: (0, 0)),
            pl.BlockSpec((D, sm), lambda i: (0, 0)),
            pl.BlockSpec((1, sm), lambda i: (0, 0)),
        ],
        out_specs=[
            pl.BlockSpec((_RB, so), lambda i: (i, 0)),
            pl.BlockSpec((1, sm), lambda i: (0, 0)),
        ],
        out_shape=[
            jax.ShapeDtypeStruct((NP, so), jnp.float32),
            jax.ShapeDtypeStruct((1, sm), jnp.float32),
        ],
        scratch_shapes=[pltpu.VMEM((1, D), jnp.float32)],
    )(p, degp, b2, wcat, bcat, wsm, bsm)


def kernel(node_features, edge_index, W1, b1, W2, b2, Wc, bc, Wh, bh, Wt, bt,
           Wp1, bp1, Wp2, bp2, Wd, bd, Ws, bs):
    n, d = node_features.shape
    e = edge_index.shape[1]
    per = NSUB * 4 * G
    etot = ((e + n + per - 1) // per) * per
    ng = etot // (NSUB * G)

    ei = edge_index.astype(jnp.int32)
    loop = jnp.arange(n, dtype=jnp.int32)
    padv = jnp.full((etot - e - n,), NP - 1, jnp.int32)
    src = jnp.concatenate([ei[0], loop, padv])
    dst = jnp.concatenate([ei[1], loop, padv])
    src3 = src.reshape(NSUB, ng, G)
    dst3 = dst.reshape(NSUB, ng, G)

    x = jnp.pad(node_features, ((0, NP - n), (0, 0)))

    ept_deg = etot // 32

    degp = _deg_call(dst, ept_deg)
    degp = degp.reshape(2, NP, 1)

    y1 = _k1_call(x, W1, degp)
    s1 = _agg_call(y1, src3, dst3, ng)
    y2 = _k2_call(s1, degp, b1.reshape(1, -1), W2)
    s2 = _agg_call(y2, src3, dst3, ng)

    wcat = jnp.concatenate([Wp1, Wp2, Wd], axis=1)
    bcat = jnp.concatenate([bp1, bp2, bd]).reshape(1, -1)
    wsm = jnp.concatenate([Wc, Wh, Wt, Ws], axis=1)
    bsm = jnp.concatenate([bc, bh, bt, bs]).reshape(1, -1)

    big, small = _k3_call(s2, degp, b2.reshape(1, -1), wcat, bcat, wsm, bsm)

    s = Wp1.shape[1]
    p1 = big[:n, :s]
    p2 = big[:n, s:2 * s]
    dep = big[:n, 2 * s:3 * s]
    value = small[:, :1]
    high = small[:, 1:5]
    mtype = small[:, 5:8]
    sel = small[:, 8:]
    return (value, high, mtype, p1, p2, dep, sel)


# speedup vs baseline: 1.2888x; 1.2805x over previous
import functools

import jax
import jax.numpy as jnp
from jax import lax
from jax.experimental import pallas as pl
from jax.experimental.pallas import tpu as pltpu
from jax.experimental.pallas import tpu_sc as plsc

N_NODES = 10000
NP = 10240
D = 128
DH = D // 2
NSUB = 16
G = 128


_MESH = plsc.VectorSubcoreMesh(core_axis_name="c", subcore_axis_name="s")
_SC_PARAMS = pltpu.CompilerParams(needs_layout_passes=False,
                                  use_tc_tiling_on_sc=False)


def _deg_body(dst_hbm, out_hbm, dstS, hist, stag, res, acc):
    c = lax.axis_index("c")
    s = lax.axis_index("s")
    wid = c * NSUB + s
    ept = dstS.shape[0]

    pltpu.sync_copy(dst_hbm.at[pl.ds(wid * ept, ept)], dstS)

    zero16 = jnp.zeros((16,), jnp.float32)

    def _zhist(i, _):
        hist[pl.ds(i * 16, 16)] = zero16
        return 0

    lax.fori_loop(0, NP // 16, _zhist, 0)

    ones16 = jnp.ones((16,), jnp.float32)

    def _histb(i, _):
        dv = dstS[pl.ds(i * 16, 16)]
        plsc.addupdate_scatter(hist, [dv], ones16)
        return 0

    lax.fori_loop(0, ept // 16, _histb, 0)

    pltpu.sync_copy(hist, acc.at[s])
    plsc.subcore_barrier()

    npt = NP // NSUB
    base = s * npt

    def _zres(i, _):
        res[pl.ds(i * 16, 16)] = zero16
        return 0

    lax.fori_loop(0, npt // 16, _zres, 0)
    for t in range(NSUB):
        pltpu.sync_copy(acc.at[t, pl.ds(base, npt)], stag)

        def _acc(i, _):
            res[pl.ds(i * 16, 16)] += stag[pl.ds(i * 16, 16)]
            return 0

        lax.fori_loop(0, npt // 16, _acc, 0)

    pltpu.sync_copy(res, out_hbm.at[c, pl.ds(base, npt)])


def _deg_call(dst_flat, ept):
    kfn = functools.partial(
        pl.kernel,
        mesh=_MESH,
        out_type=jax.ShapeDtypeStruct((2, NP), jnp.float32),
        scratch_types=[
            pltpu.VMEM((ept,), jnp.int32),
            pltpu.VMEM((NP,), jnp.float32),
            pltpu.VMEM((NP // NSUB,), jnp.float32),
            pltpu.VMEM((NP // NSUB,), jnp.float32),
            pltpu.VMEM_SHARED((NSUB, NP), jnp.float32),
        ],
        compiler_params=_SC_PARAMS,
    )(_deg_body)
    return kfn(dst_flat)


def _agg_body(y_hbm, src_hbm, dst_hbm, out_hbm, srcS, dstS, buf0, buf1, sem0, sem1, acc):
    c = lax.axis_index("c")
    s = lax.axis_index("s")
    ng = srcS.shape[0]
    rows_per_tile = NP // NSUB
    wb = 128
    nchunk = rows_per_tile // wb
    bufs = (buf0, buf1)
    sems = (sem0, sem1)

    zero16 = jnp.zeros((16,), jnp.float32)

    def _zb(i, _):
        buf0[i, pl.ds(0, 16)] = zero16
        buf0[i, pl.ds(16, 16)] = zero16
        buf0[i, pl.ds(32, 16)] = zero16
        buf0[i, pl.ds(48, 16)] = zero16
        return 0

    lax.fori_loop(0, wb, _zb, 0)
    for k in range(nchunk):
        pltpu.sync_copy(buf0.at[pl.ds(0, wb)],
                        acc.at[pl.ds(s * rows_per_tile + k * wb, wb)])

    pltpu.sync_copy(src_hbm.at[s], srcS)
    pltpu.sync_copy(dst_hbm.at[s], dstS)
    plsc.subcore_barrier()

    def _body(t, _):
        j0 = 2 * t
        j1 = 2 * t + 1
        cp0 = pltpu.async_copy(y_hbm.at[c].at[srcS.at[j0]], buf0, sem0)
        cp1 = pltpu.async_copy(y_hbm.at[c].at[srcS.at[j1]], buf1, sem1)
        cp0.wait()
        pltpu.sync_copy(buf0, acc.at[dstS.at[j0]], add=True)
        cp1.wait()
        pltpu.sync_copy(buf1, acc.at[dstS.at[j1]], add=True)
        return 0

    lax.fori_loop(0, ng // 2, _body, 0)

    plsc.subcore_barrier()
    for k in range(nchunk):
        pltpu.sync_copy(acc.at[pl.ds(s * rows_per_tile + k * wb, wb)],
                        buf0.at[pl.ds(0, wb)])
        pltpu.sync_copy(buf0.at[pl.ds(0, wb)],
                        out_hbm.at[c, pl.ds(s * rows_per_tile + k * wb, wb)])


def _agg_call(y, src3, dst3, ng):
    kfn = functools.partial(
        pl.kernel,
        mesh=_MESH,
        out_type=jax.ShapeDtypeStruct((2, NP, DH), jnp.float32),
        scratch_types=[
            pltpu.VMEM((ng, G), jnp.int32),
            pltpu.VMEM((ng, G), jnp.int32),
            pltpu.VMEM((G, DH), jnp.float32),
            pltpu.VMEM((G, DH), jnp.float32),
            pltpu.SemaphoreType.DMA,
            pltpu.SemaphoreType.DMA,
            pltpu.VMEM_SHARED((NP, DH), jnp.float32),
        ],
        compiler_params=_SC_PARAMS,
    )(_agg_body)
    return kfn(y, src3, dst3)


_RB = 1280


def _dinv_block(degp):
    deg = degp[0] + degp[1]
    return lax.rsqrt(jnp.maximum(deg, 1.0))


def _split_store(o_ref, y):
    o_ref[0] = y[:, :DH]
    o_ref[1] = y[:, DH:]


def _k1_body(x_ref, w_ref, degp_ref, o_ref):
    dinv = _dinv_block(degp_ref[...])
    y = jnp.dot(x_ref[...], w_ref[...], preferred_element_type=jnp.float32)
    _split_store(o_ref, y * dinv)


def _k1_call(x, w1, degp):
    grid = NP // _RB
    return pl.pallas_call(
        _k1_body,
        grid=(grid,),
        in_specs=[
            pl.BlockSpec((_RB, D), lambda i: (i, 0)),
            pl.BlockSpec((D, D), lambda i: (0, 0)),
            pl.BlockSpec((2, _RB, 1), lambda i: (0, i, 0)),
        ],
        out_specs=pl.BlockSpec((2, _RB, DH), lambda i: (0, i, 0)),
        out_shape=jax.ShapeDtypeStruct((2, NP, DH), jnp.float32),
    )(x, w1, degp)


def _k2_body(p_ref, degp_ref, b_ref, w_ref, o_ref):
    i = pl.program_id(0)
    dinv = _dinv_block(degp_ref[...])
    sfull = jnp.concatenate([p_ref[0], p_ref[1]], axis=1)
    h = jax.nn.relu(sfull * dinv + b_ref[...])
    y = jnp.dot(h, w_ref[...], preferred_element_type=jnp.float32) * dinv
    rows = i * _RB + lax.broadcasted_iota(jnp.int32, (_RB, 1), 0)
    _split_store(o_ref, jnp.where(rows < N_NODES, y, 0.0))


def _k2_call(p, degp, b1, w2):
    grid = NP // _RB
    return pl.pallas_call(
        _k2_body,
        grid=(grid,),
        in_specs=[
            pl.BlockSpec((2, _RB, DH), lambda i: (0, i, 0)),
            pl.BlockSpec((2, _RB, 1), lambda i: (0, i, 0)),
            pl.BlockSpec((1, D), lambda i: (0, 0)),
            pl.BlockSpec((D, D), lambda i: (0, 0)),
        ],
        out_specs=pl.BlockSpec((2, _RB, DH), lambda i: (0, i, 0)),
        out_shape=jax.ShapeDtypeStruct((2, NP, DH), jnp.float32),
    )(p, degp, b1, w2)


def _k3_body(p_ref, degp_ref, b2_ref, wcat_ref, bcat_ref, wsm_ref,
             bsm_ref, big_ref, small_ref, gacc_ref):
    i = pl.program_id(0)
    nsteps = pl.num_programs(0)
    dinv = _dinv_block(degp_ref[...])
    sfull = jnp.concatenate([p_ref[0], p_ref[1]], axis=1)
    h = jax.nn.relu(sfull * dinv + b2_ref[...])
    rows = i * _RB + lax.broadcasted_iota(jnp.int32, (_RB, 1), 0)
    hm = jnp.where(rows < N_NODES, h, 0.0)

    @pl.when(i == 0)
    def _():
        gacc_ref[...] = jnp.zeros_like(gacc_ref)

    gacc_ref[...] += jnp.sum(hm, axis=0, keepdims=True)

    big_ref[...] = (
        jnp.dot(h, wcat_ref[...], preferred_element_type=jnp.float32)
        + bcat_ref[...]
    )

    @pl.when(i == nsteps - 1)
    def _():
        g = gacc_ref[...] * (1.0 / N_NODES)
        small_ref[...] = (
            jnp.dot(g, wsm_ref[...], preferred_element_type=jnp.float32)
            + bsm_ref[...]
        )


def _k3_call(p, degp, b2, wcat, bcat, wsm, bsm):
    grid = NP // _RB
    so = wcat.shape[1]
    sm = wsm.shape[1]
    return pl.pallas_call(
        _k3_body,
        grid=(grid,),
        in_specs=[
            pl.BlockSpec((2, _RB, DH), lambda i: (0, i, 0)),
            pl.BlockSpec((2, _RB, 1), lambda i: (0, i, 0)),
            pl.BlockSpec((1, D), lambda i: (0, 0)),
            pl.BlockSpec((D, so), lambda i: (0, 0)),
            pl.BlockSpec((1, so), lambda i: (0, 0)),
            pl.BlockSpec((D, sm), lambda i: (0, 0)),
            pl.BlockSpec((1, sm), lambda i: (0, 0)),
        ],
        out_specs=[
            pl.BlockSpec((_RB, so), lambda i: (i, 0)),
            pl.BlockSpec((1, sm), lambda i: (0, 0)),
        ],
        out_shape=[
            jax.ShapeDtypeStruct((NP, so), jnp.float32),
            jax.ShapeDtypeStruct((1, sm), jnp.float32),
        ],
        scratch_shapes=[pltpu.VMEM((1, D), jnp.float32)],
    )(p, degp, b2, wcat, bcat, wsm, bsm)


def kernel(node_features, edge_index, W1, b1, W2, b2, Wc, bc, Wh, bh, Wt, bt,
           Wp1, bp1, Wp2, bp2, Wd, bd, Ws, bs):
    n, d = node_features.shape
    e = edge_index.shape[1]
    per = NSUB * 2 * G
    etot = ((e + n + per - 1) // per) * per
    ng = etot // (NSUB * G)

    ei = edge_index.astype(jnp.int32)
    loop = jnp.arange(n, dtype=jnp.int32)
    padv = jnp.full((etot - e - n,), NP - 1, jnp.int32)
    src = jnp.concatenate([ei[0], loop, padv])
    dst = jnp.concatenate([ei[1], loop, padv])
    src3 = src.reshape(NSUB, ng, G)
    dst3 = dst.reshape(NSUB, ng, G)

    x = jnp.pad(node_features, ((0, NP - n), (0, 0)))

    ept_deg = etot // 32

    degp = _deg_call(dst, ept_deg)
    degp = degp.reshape(2, NP, 1)

    y1 = _k1_call(x, W1, degp)
    s1 = _agg_call(y1, src3, dst3, ng)
    y2 = _k2_call(s1, degp, b1.reshape(1, -1), W2)
    s2 = _agg_call(y2, src3, dst3, ng)

    wcat = jnp.concatenate([Wp1, Wp2, Wd], axis=1)
    bcat = jnp.concatenate([bp1, bp2, bd]).reshape(1, -1)
    wsm = jnp.concatenate([Wc, Wh, Wt, Ws], axis=1)
    bsm = jnp.concatenate([bc, bh, bt, bs]).reshape(1, -1)

    big, small = _k3_call(s2, degp, b2.reshape(1, -1), wcat, bcat, wsm, bsm)

    s = Wp1.shape[1]
    p1 = big[:n, :s]
    p2 = big[:n, s:2 * s]
    dep = big[:n, 2 * s:3 * s]
    value = small[:, :1]
    high = small[:, 1:5]
    mtype = small[:, 5:8]
    sel = small[:, 8:]
    return (value, high, mtype, p1, p2, dep, sel)


# speedup vs baseline: 1.3389x; 1.0389x over previous
import functools

import jax
import jax.numpy as jnp
from jax import lax
from jax.experimental import pallas as pl
from jax.experimental.pallas import tpu as pltpu
from jax.experimental.pallas import tpu_sc as plsc

N_NODES = 10000
NP = 10240
D = 128
DH = D // 2
NSUB = 16
G = 128


_MESH = plsc.VectorSubcoreMesh(core_axis_name="c", subcore_axis_name="s")
_SC_PARAMS = pltpu.CompilerParams(needs_layout_passes=False,
                                  use_tc_tiling_on_sc=False)


def _deg_body(dst_hbm, out_hbm, dstS, hist, stag, res, acc):
    c = lax.axis_index("c")
    s = lax.axis_index("s")
    wid = c * NSUB + s
    ept = dstS.shape[0]

    pltpu.sync_copy(dst_hbm.at[pl.ds(wid * ept, ept)], dstS)

    zero16 = jnp.zeros((16,), jnp.float32)

    def _zhist(i, _):
        hist[pl.ds(i * 16, 16)] = zero16
        return 0

    lax.fori_loop(0, NP // 16, _zhist, 0)

    ones16 = jnp.ones((16,), jnp.float32)

    def _histb(i, _):
        dv = dstS[pl.ds(i * 16, 16)]
        plsc.addupdate_scatter(hist, [dv], ones16)
        return 0

    lax.fori_loop(0, ept // 16, _histb, 0)

    pltpu.sync_copy(hist, acc.at[s])
    plsc.subcore_barrier()

    npt = NP // NSUB
    base = s * npt

    def _zres(i, _):
        res[pl.ds(i * 16, 16)] = zero16
        return 0

    lax.fori_loop(0, npt // 16, _zres, 0)
    for t in range(NSUB):
        pltpu.sync_copy(acc.at[t, pl.ds(base, npt)], stag)

        def _acc(i, _):
            res[pl.ds(i * 16, 16)] += stag[pl.ds(i * 16, 16)]
            return 0

        lax.fori_loop(0, npt // 16, _acc, 0)

    pltpu.sync_copy(res, out_hbm.at[c, pl.ds(base, npt)])


def _deg_call(dst_flat, ept):
    kfn = functools.partial(
        pl.kernel,
        mesh=_MESH,
        out_type=jax.ShapeDtypeStruct((2, NP), jnp.float32),
        scratch_types=[
            pltpu.VMEM((ept,), jnp.int32),
            pltpu.VMEM((NP,), jnp.float32),
            pltpu.VMEM((NP // NSUB,), jnp.float32),
            pltpu.VMEM((NP // NSUB,), jnp.float32),
            pltpu.VMEM_SHARED((NSUB, NP), jnp.float32),
        ],
        compiler_params=_SC_PARAMS,
    )(_deg_body)
    return kfn(dst_flat)


def _agg_body(y_hbm, src_hbm, dst_hbm, out_hbm, srcS, dstS, buf0, buf1, sem0, sem1, acc):
    c = lax.axis_index("c")
    s = lax.axis_index("s")
    ng = srcS.shape[0]
    rows_per_tile = NP // NSUB
    wb = 128
    nchunk = rows_per_tile // wb
    bufs = (buf0, buf1)
    sems = (sem0, sem1)

    zero16 = jnp.zeros((16,), jnp.float32)

    def _zb(i, _):
        buf0[i, pl.ds(0, 16)] = zero16
        buf0[i, pl.ds(16, 16)] = zero16
        buf0[i, pl.ds(32, 16)] = zero16
        buf0[i, pl.ds(48, 16)] = zero16
        return 0

    lax.fori_loop(0, wb, _zb, 0)
    for k in range(nchunk):
        pltpu.sync_copy(buf0.at[pl.ds(0, wb)],
                        acc.at[pl.ds(s * rows_per_tile + k * wb, wb)])

    pltpu.sync_copy(src_hbm.at[s], srcS)
    pltpu.sync_copy(dst_hbm.at[s], dstS)
    plsc.subcore_barrier()

    def _body(t, _):
        j0 = 2 * t
        j1 = 2 * t + 1
        cp0 = pltpu.async_copy(y_hbm.at[c].at[srcS.at[j0]], buf0, sem0)
        cp1 = pltpu.async_copy(y_hbm.at[c].at[srcS.at[j1]], buf1, sem1)
        cp0.wait()
        pltpu.sync_copy(buf0, acc.at[dstS.at[j0]], add=True)
        cp1.wait()
        pltpu.sync_copy(buf1, acc.at[dstS.at[j1]], add=True)
        return 0

    lax.fori_loop(0, ng // 2, _body, 0)

    plsc.subcore_barrier()
    for k in range(nchunk):
        pltpu.sync_copy(acc.at[pl.ds(s * rows_per_tile + k * wb, wb)],
                        buf0.at[pl.ds(0, wb)])
        pltpu.sync_copy(buf0.at[pl.ds(0, wb)],
                        out_hbm.at[c, pl.ds(s * rows_per_tile + k * wb, wb)])


def _agg_call(y, src3, dst3, ng):
    kfn = functools.partial(
        pl.kernel,
        mesh=_MESH,
        out_type=jax.ShapeDtypeStruct((2, NP, DH), jnp.float32),
        scratch_types=[
            pltpu.VMEM((ng, G), jnp.int32),
            pltpu.VMEM((ng, G), jnp.int32),
            pltpu.VMEM((G, DH), jnp.float32),
            pltpu.VMEM((G, DH), jnp.float32),
            pltpu.SemaphoreType.DMA,
            pltpu.SemaphoreType.DMA,
            pltpu.VMEM_SHARED((NP, DH), jnp.float32),
        ],
        compiler_params=_SC_PARAMS,
    )(_agg_body)
    return kfn(y, src3, dst3)


_RB = 1280


def _dinv_block(degp):
    deg = degp[0] + degp[1]
    return lax.rsqrt(jnp.maximum(deg, 1.0))


def _split_store(o_ref, y):
    o_ref[0] = y[:, :DH]
    o_ref[1] = y[:, DH:]


def _k1_body(x_ref, w_ref, degp_ref, o_ref):
    dinv = _dinv_block(degp_ref[...])
    y = jnp.dot(x_ref[...], w_ref[...], preferred_element_type=jnp.float32)
    _split_store(o_ref, y * dinv)


def _k1_call(x, w1, degp):
    grid = NP // _RB
    return pl.pallas_call(
        _k1_body,
        grid=(grid,),
        in_specs=[
            pl.BlockSpec((_RB, D), lambda i: (i, 0)),
            pl.BlockSpec((D, D), lambda i: (0, 0)),
            pl.BlockSpec((2, _RB, 1), lambda i: (0, i, 0)),
        ],
        out_specs=pl.BlockSpec((2, _RB, DH), lambda i: (0, i, 0)),
        out_shape=jax.ShapeDtypeStruct((2, NP, DH), jnp.float32),
    )(x, w1, degp)


def _k2_body(p_ref, degp_ref, b_ref, w_ref, o_ref):
    i = pl.program_id(0)
    dinv = _dinv_block(degp_ref[...])
    sfull = jnp.concatenate([p_ref[0], p_ref[1]], axis=1)
    h = jax.nn.relu(sfull * dinv + b_ref[...])
    y = jnp.dot(h, w_ref[...], preferred_element_type=jnp.float32) * dinv
    rows = i * _RB + lax.broadcasted_iota(jnp.int32, (_RB, 1), 0)
    _split_store(o_ref, jnp.where(rows < N_NODES, y, 0.0))


def _k2_call(p, degp, b1, w2):
    grid = NP // _RB
    return pl.pallas_call(
        _k2_body,
        grid=(grid,),
        in_specs=[
            pl.BlockSpec((2, _RB, DH), lambda i: (0, i, 0)),
            pl.BlockSpec((2, _RB, 1), lambda i: (0, i, 0)),
            pl.BlockSpec((1, D), lambda i: (0, 0)),
            pl.BlockSpec((D, D), lambda i: (0, 0)),
        ],
        out_specs=pl.BlockSpec((2, _RB, DH), lambda i: (0, i, 0)),
        out_shape=jax.ShapeDtypeStruct((2, NP, DH), jnp.float32),
    )(p, degp, b1, w2)


def _k3_body(p_ref, degp_ref, b2_ref, wcat_ref, bcat_ref, wsm_ref,
             bsm_ref, big1_ref, big2_ref, big3_ref, small_ref, gacc_ref):
    i = pl.program_id(0)
    nsteps = pl.num_programs(0)
    dinv = _dinv_block(degp_ref[...])
    sfull = jnp.concatenate([p_ref[0], p_ref[1]], axis=1)
    h = jax.nn.relu(sfull * dinv + b2_ref[...])
    rows = i * _RB + lax.broadcasted_iota(jnp.int32, (_RB, 1), 0)
    hm = jnp.where(rows < N_NODES, h, 0.0)

    @pl.when(i == 0)
    def _():
        gacc_ref[...] = jnp.zeros_like(gacc_ref)

    gacc_ref[...] += jnp.sum(hm, axis=0, keepdims=True)

    big = (
        jnp.dot(h, wcat_ref[...], preferred_element_type=jnp.float32)
        + bcat_ref[...]
    )
    so = big.shape[1] // 3
    big1_ref[...] = big[:, :so]
    big2_ref[...] = big[:, so:2 * so]
    big3_ref[...] = big[:, 2 * so:]

    @pl.when(i == nsteps - 1)
    def _():
        g = gacc_ref[...] * (1.0 / N_NODES)
        small_ref[...] = (
            jnp.dot(g, wsm_ref[...], preferred_element_type=jnp.float32)
            + bsm_ref[...]
        )


def _k3_call(p, degp, b2, wcat, bcat, wsm, bsm):
    grid = NP // _RB
    so = wcat.shape[1]
    sm = wsm.shape[1]
    return pl.pallas_call(
        _k3_body,
        grid=(grid,),
        in_specs=[
            pl.BlockSpec((2, _RB, DH), lambda i: (0, i, 0)),
            pl.BlockSpec((2, _RB, 1), lambda i: (0, i, 0)),
            pl.BlockSpec((1, D), lambda i: (0, 0)),
            pl.BlockSpec((D, so), lambda i: (0, 0)),
            pl.BlockSpec((1, so), lambda i: (0, 0)),
            pl.BlockSpec((D, sm), lambda i: (0, 0)),
            pl.BlockSpec((1, sm), lambda i: (0, 0)),
        ],
        out_specs=[
            pl.BlockSpec((_RB, so // 3), lambda i: (i, 0)),
            pl.BlockSpec((_RB, so // 3), lambda i: (i, 0)),
            pl.BlockSpec((_RB, so // 3), lambda i: (i, 0)),
            pl.BlockSpec((1, sm), lambda i: (0, 0)),
        ],
        out_shape=[
            jax.ShapeDtypeStruct((N_NODES, so // 3), jnp.float32),
            jax.ShapeDtypeStruct((N_NODES, so // 3), jnp.float32),
            jax.ShapeDtypeStruct((N_NODES, so // 3), jnp.float32),
            jax.ShapeDtypeStruct((1, sm), jnp.float32),
        ],
        scratch_shapes=[pltpu.VMEM((1, D), jnp.float32)],
    )(p, degp, b2, wcat, bcat, wsm, bsm)


def kernel(node_features, edge_index, W1, b1, W2, b2, Wc, bc, Wh, bh, Wt, bt,
           Wp1, bp1, Wp2, bp2, Wd, bd, Ws, bs):
    n, d = node_features.shape
    e = edge_index.shape[1]
    per = NSUB * 2 * G
    etot = ((e + n + per - 1) // per) * per
    ng = etot // (NSUB * G)

    ei = edge_index.astype(jnp.int32)
    loop = jnp.arange(n, dtype=jnp.int32)
    padv = jnp.full((etot - e - n,), NP - 1, jnp.int32)
    src = jnp.concatenate([ei[0], loop, padv])
    dst = jnp.concatenate([ei[1], loop, padv])
    src3 = src.reshape(NSUB, ng, G)
    dst3 = dst.reshape(NSUB, ng, G)

    x = jnp.pad(node_features, ((0, NP - n), (0, 0)))

    ept_deg = etot // 32

    degp = _deg_call(dst, ept_deg)
    degp = degp.reshape(2, NP, 1)

    y1 = _k1_call(x, W1, degp)
    s1 = _agg_call(y1, src3, dst3, ng)
    y2 = _k2_call(s1, degp, b1.reshape(1, -1), W2)
    s2 = _agg_call(y2, src3, dst3, ng)

    wcat = jnp.concatenate([Wp1, Wp2, Wd], axis=1)
    bcat = jnp.concatenate([bp1, bp2, bd]).reshape(1, -1)
    wsm = jnp.concatenate([Wc, Wh, Wt, Ws], axis=1)
    bsm = jnp.concatenate([bc, bh, bt, bs]).reshape(1, -1)

    p1, p2, dep, small = _k3_call(s2, degp, b2.reshape(1, -1), wcat, bcat,
                                  wsm, bsm)

    value = small[:, :1]
    high = small[:, 1:5]
    mtype = small[:, 5:8]
    sel = small[:, 8:]
    return (value, high, mtype, p1, p2, dep, sel)
